# Initial kernel scaffold; baseline (speedup 1.0000x reference)
import jax
import jax.numpy as jnp
from jax.experimental import pallas as pl


def kernel(hidden_states, w_gate, w1, w2, w3):
    raise NotImplementedError("write your pallas kernel here")


# speedup vs baseline: 1.3778x; 1.3778x over previous
import functools

import jax
import jax.numpy as jnp
from jax import lax
from jax.experimental import pallas as pl
from jax.experimental.pallas import tpu as pltpu

NUM_EXPERTS = 8
HIDDEN = 1024
INTERMEDIATE = 4096
TOKENS = 2048
JITTER_EPS = 0.01
EPAD = 128

N_CHUNK = 512
NI = INTERMEDIATE // N_CHUNK


def _routing_kernel(scores_ref, wcomb_ref):
    scores = scores_ref[...]
    ids = lax.broadcasted_iota(jnp.int32, scores.shape, 1)
    neg_inf = jnp.float32(-jnp.inf)

    max1 = jnp.max(scores, axis=-1, keepdims=True)
    is_max1 = scores == max1
    ind1 = jnp.min(jnp.where(is_max1, ids, NUM_EXPERTS), axis=-1, keepdims=True)
    factor = jnp.maximum(jnp.abs(scores), max1)
    mask1 = (max1 - scores) / factor > 2.0 * JITTER_EPS
    mg1 = jnp.where(mask1, neg_inf, scores)
    p1 = jnp.exp(mg1 - max1)
    p1sum = jnp.sum(p1, axis=-1, keepdims=True)
    oh1 = ids == ind1
    mult1 = jnp.sum(jnp.where(oh1, p1, 0.0), axis=-1, keepdims=True) / p1sum

    masked_scores = jnp.where(oh1, neg_inf, scores)
    max2 = jnp.max(masked_scores, axis=-1, keepdims=True)
    is_max2 = masked_scores == max2
    ind2 = jnp.min(jnp.where(is_max2, ids, NUM_EXPERTS), axis=-1, keepdims=True)
    factor2 = jnp.maximum(jnp.abs(scores), max2)
    mask2 = (max2 - scores) / factor2 > 2.0 * JITTER_EPS
    mg2 = jnp.where(mask2, neg_inf, masked_scores)
    p2 = jnp.exp(mg2 - max2)
    p2sum = jnp.sum(p2, axis=-1, keepdims=True)
    oh2 = ids == ind2
    mult2 = jnp.sum(jnp.where(oh2, p2, 0.0), axis=-1, keepdims=True) / p2sum

    idsp = lax.broadcasted_iota(jnp.int32, (TOKENS, EPAD), 1)
    wcomb = jnp.where(idsp == ind1, mult1, 0.0) + jnp.where(idsp == ind2, mult2, 0.0)
    wcomb_ref[...] = wcomb


def _routing(x, w_gate):
    scores = x @ w_gate.T
    return pl.pallas_call(
        _routing_kernel,
        out_shape=jax.ShapeDtypeStruct((TOKENS, EPAD), jnp.float32),
    )(scores)


def _dense_expert_kernel(x_ref, wcomb_ref, w1_ref, w3_ref, w2_ref, out_ref):
    e = pl.program_id(0)
    n = pl.program_id(1)

    @pl.when(jnp.logical_and(e == 0, n == 0))
    def _init():
        out_ref[...] = jnp.zeros_like(out_ref)

    x = x_ref[...]
    w1 = w1_ref[0]
    w3 = w3_ref[0]
    w2 = w2_ref[0]
    gate = lax.dot_general(x, w1, (((1,), (1,)), ((), ())),
                           preferred_element_type=jnp.float32)
    up = lax.dot_general(x, w3, (((1,), (1,)), ((), ())),
                         preferred_element_type=jnp.float32)
    h = gate * jax.nn.sigmoid(gate) * up
    lane = lax.broadcasted_iota(jnp.int32, wcomb_ref.shape, 1)
    wcol = jnp.sum(jnp.where(lane == e, wcomb_ref[...], 0.0), axis=-1,
                   keepdims=True)
    hw = h * wcol
    out_ref[...] += lax.dot_general(hw, w2, (((1,), (1,)), ((), ())),
                                    preferred_element_type=jnp.float32)


def _dense_experts(x, wcomb, w1, w2, w3):
    return pl.pallas_call(
        _dense_expert_kernel,
        grid=(NUM_EXPERTS, NI),
        in_specs=[
            pl.BlockSpec((TOKENS, HIDDEN), lambda e, n: (0, 0)),
            pl.BlockSpec((TOKENS, EPAD), lambda e, n: (0, 0)),
            pl.BlockSpec((1, N_CHUNK, HIDDEN), lambda e, n: (e, n, 0)),
            pl.BlockSpec((1, N_CHUNK, HIDDEN), lambda e, n: (e, n, 0)),
            pl.BlockSpec((1, HIDDEN, N_CHUNK), lambda e, n: (e, 0, n)),
        ],
        out_specs=pl.BlockSpec((TOKENS, HIDDEN), lambda e, n: (0, 0)),
        out_shape=jax.ShapeDtypeStruct((TOKENS, HIDDEN), jnp.float32),
        compiler_params=pltpu.CompilerParams(
            dimension_semantics=("arbitrary", "arbitrary"),
        ),
    )(x, wcomb, w1, w3, w2)


def kernel(hidden_states, w_gate, w1, w2, w3):
    x = hidden_states.reshape(-1, HIDDEN)
    wcomb = _routing(x, w_gate)
    out = _dense_experts(x, wcomb, w1, w2, w3)
    return out.reshape(hidden_states.shape)
